# Initial kernel scaffold; baseline (speedup 1.0000x reference)
import jax
import jax.numpy as jnp
from jax.experimental import pallas as pl


def kernel(x, W, b):
    raise NotImplementedError("write your pallas kernel here")


# speedup vs baseline: 9.5635x; 9.5635x over previous
import jax
import jax.numpy as jnp
from jax.experimental import pallas as pl

_K_LEVELS = (32, 64, 128)
_ROWS_PER_BLOCK = 256
_D = 768
_H = 2048


def _f32_sort_key(h):
    i = jax.lax.bitcast_convert_type(h, jnp.int32)
    return jnp.where(i < 0, i ^ jnp.int32(0x7FFFFFFF), i)


def _kth_largest_key(key, k, iters=32):
    lo = jnp.min(key, axis=1, keepdims=True)
    hi = jnp.max(key, axis=1, keepdims=True)

    def body(_, lh):
        lo, hi = lh
        mid = (lo & hi) + ((lo ^ hi) >> 1) + ((lo ^ hi) & 1)
        cnt = jnp.sum((key >= mid).astype(jnp.int32), axis=1, keepdims=True)
        ge = cnt >= k
        lo = jnp.where(ge, mid, lo)
        hi = jnp.where(ge, hi, mid - 1)
        return lo, hi

    lo, hi = jax.lax.fori_loop(0, iters, body, (lo, hi))
    return lo


def _encoder_block(x_ref, wt_ref, b_ref, o32_ref, o64_ref, o128_ref):
    h = jnp.dot(x_ref[...], wt_ref[...], preferred_element_type=jnp.float32)
    h = h + b_ref[...]
    key = _f32_sort_key(h)
    relu_h = jnp.maximum(h, 0.0)
    for k, o_ref in zip(_K_LEVELS, (o32_ref, o64_ref, o128_ref)):
        t = _kth_largest_key(key, k)
        o_ref[...] = jnp.where(key >= t, relu_h, 0.0)


def kernel(x, W, b):
    n = x.shape[0]
    wt = W.T.astype(jnp.float32)
    b2 = b.reshape(1, _H)
    outs = pl.pallas_call(
        _encoder_block,
        grid=(n // _ROWS_PER_BLOCK,),
        in_specs=[
            pl.BlockSpec((_ROWS_PER_BLOCK, _D), lambda i: (i, 0)),
            pl.BlockSpec((_D, _H), lambda i: (0, 0)),
            pl.BlockSpec((1, _H), lambda i: (0, 0)),
        ],
        out_specs=[pl.BlockSpec((_ROWS_PER_BLOCK, _H), lambda i: (i, 0))] * 3,
        out_shape=[jax.ShapeDtypeStruct((n, _H), jnp.float32)] * 3,
    )(x, wt, b2)
    return tuple(outs)
